# baseline (speedup 1.0000x reference)
import functools

import jax
import jax.numpy as jnp
from jax import lax
from jax.experimental import pallas as pl
from jax.experimental.pallas import tpu as pltpu
from jax.experimental.pallas import tpu_sc as plsc

NC, NS = 2, 16
NW = NC * NS
N = 10000
H = 128
E = 320000
L = 100000
CH = 128
KPT = 162
SEXT = KPT // 6
NCHP = KPT * NS
PE = NCHP * CH
NA = N + 8
LP = 100096
LCHUNK = LP // CH
K_DEC = -(-LCHUNK // NW)
RC = 400
NRC = N // RC
K_RC = -(-NRC // NS)

_MESH = plsc.VectorSubcoreMesh(
    core_axis_name="c", subcore_axis_name="s", num_cores=NC, num_subcores=NS)


def _seg_pair_body(idx_a, tbl_a, init_a, idx_b, tbl_b, init_b,
                   out_a, out_b, *scr):
    rows = scr[0:3]
    idx = scr[3:9]
    acc = scr[9]
    semg = scr[10:13]
    sems = scr[13:16]
    semi = scr[16:22]
    c = lax.axis_index("c")
    t = lax.axis_index("s")

    def rows_loop(body_fn):
        def body(k, carry):
            cid = k * NS + t

            @pl.when(cid < NRC)
            def _():
                body_fn(cid * RC)

            return carry

        lax.fori_loop(0, K_RC, body, 0)

    def run(idx_h, tbl_h, init_h, out_h):
        def load_idx(k, m):
            pltpu.async_copy(idx_h.at[t * KPT + k], idx[m], semi[m])

        def wait_idx(k, m):
            pltpu.make_async_copy(
                idx_h.at[t * KPT + k], idx[m], semi[m]).wait()

        def gather(m, j):
            pltpu.async_copy(tbl_h.at[idx[m].at[0]], rows[j], semg[j])

        def wait_g(m, j):
            pltpu.make_async_copy(
                tbl_h.at[idx[m].at[0]], rows[j], semg[j]).wait()

        def scat(m, j):
            pltpu.async_copy(rows[j], acc.at[idx[m].at[1]], sems[j],
                             add=True)

        def wait_s(m, j):
            pltpu.make_async_copy(
                rows[j], acc.at[idx[m].at[1]], sems[j]).wait()

        for m in range(3):
            load_idx(m, m)
        rows_loop(lambda r0: pltpu.sync_copy(
            init_h.at[pl.ds(r0, RC)], acc.at[pl.ds(r0, RC)]))
        plsc.subcore_barrier()

        def sextet(kk, carry):
            k0 = 6 * kk
            for j in range(6):
                k = k0 + j
                jb = j % 3
                mprev = (j + 3) % 6
                if j >= 3:
                    wait_s(mprev, jb)
                else:
                    @pl.when(kk > 0)
                    def _():
                        wait_s(mprev, jb)

                @pl.when(k + 3 < KPT)
                def _():
                    load_idx(k + 3, mprev)

                wait_idx(k, j)
                gather(j, jb)
                wait_g(j, jb)
                scat(j, jb)
            return carry

        lax.fori_loop(0, SEXT, sextet, 0)
        for k in range(KPT - 3, KPT):
            wait_s(k % 6, k % 3)
        plsc.subcore_barrier()
        rows_loop(lambda r0: pltpu.sync_copy(
            acc.at[pl.ds(r0, RC)], out_h.at[pl.ds(r0, RC)]))

    @pl.when(c == 0)
    def _():
        run(idx_a, tbl_a, init_a, out_a)

    @pl.when(c == 1)
    def _():
        run(idx_b, tbl_b, init_b, out_b)


_seg_pair = pl.kernel(
    _seg_pair_body,
    out_type=(jax.ShapeDtypeStruct((N, H), jnp.float32),
              jax.ShapeDtypeStruct((N, H), jnp.float32)),
    mesh=_MESH,
    scratch_types=(
        [pltpu.VMEM((CH, H), jnp.float32)] * 3
        + [pltpu.VMEM((2, CH), jnp.int32)] * 6
        + [pltpu.VMEM_SHARED((NA, H), jnp.float32)]
        + [pltpu.SemaphoreType.DMA] * 12
    ),
)


def _dec_gather_body(ud, up, ridx_h, cidx_h, s_h, idx_v, buf, sem):
    wid = lax.axis_index("c") * NS + lax.axis_index("s")

    def body(k, carry):
        cid = k * NW + wid

        @pl.when(cid < LCHUNK)
        def _():
            base = cid * CH
            pltpu.sync_copy(ridx_h.at[pl.ds(base, CH)], idx_v)
            pltpu.async_copy(ud.at[idx_v], buf, sem).wait()
            pltpu.sync_copy(cidx_h.at[pl.ds(base, CH)], idx_v)
            pltpu.async_copy(up.at[idx_v], buf, sem, add=True).wait()
            pltpu.sync_copy(buf, s_h.at[pl.ds(base, CH)])

        return carry

    lax.fori_loop(0, K_DEC, body, 0)


_dec_gather = pl.kernel(
    _dec_gather_body,
    out_type=jax.ShapeDtypeStruct((LP, H), jnp.float32),
    mesh=_MESH,
    scratch_types=[
        pltpu.VMEM((CH,), jnp.int32),
        pltpu.VMEM((CH, H), jnp.float32),
        pltpu.SemaphoreType.DMA,
    ],
)


def _quad_body(a_ref, b_ref, w1, w2, w3, w4, bias1, bias2,
               o1, o2, o3, o4, *, relu):
    a = a_ref[...]
    b = b_ref[...]
    if relu:
        a = jnp.maximum(a, 0.0)
        b = jnp.maximum(b, 0.0)
    f32 = jnp.float32
    o1[...] = jnp.dot(a, w1[...], preferred_element_type=f32)
    o2[...] = jnp.dot(b, w2[...], preferred_element_type=f32) + bias1[...]
    o3[...] = jnp.dot(b, w3[...], preferred_element_type=f32)
    o4[...] = jnp.dot(a, w4[...], preferred_element_type=f32) + bias2[...]


def _make_quad(relu):
    blk = 1000
    grid = N // blk
    row_spec = pl.BlockSpec((blk, H), lambda i: (i, 0))
    full_spec = pl.BlockSpec((H, H), lambda i: (0, 0))
    bias_spec = pl.BlockSpec((1, H), lambda i: (0, 0))
    return pl.pallas_call(
        functools.partial(_quad_body, relu=relu),
        grid=(grid,),
        in_specs=[row_spec, row_spec, full_spec, full_spec, full_spec,
                  full_spec, bias_spec, bias_spec],
        out_specs=[row_spec, row_spec, row_spec, row_spec],
        out_shape=[jax.ShapeDtypeStruct((N, H), jnp.float32)] * 4,
    )


_quad_plain = _make_quad(relu=False)
_quad_relu = _make_quad(relu=True)


def _dual_body(a_ref, b_ref, w1, w2, bias1, o1, o2):
    f32 = jnp.float32
    o1[...] = jnp.dot(a_ref[...], w1[...], preferred_element_type=f32) + bias1[...]
    o2[...] = jnp.dot(b_ref[...], w2[...], preferred_element_type=f32)


def _make_dual():
    blk = 1000
    grid = N // blk
    row_spec = pl.BlockSpec((blk, H), lambda i: (i, 0))
    full_spec = pl.BlockSpec((H, H), lambda i: (0, 0))
    bias_spec = pl.BlockSpec((1, H), lambda i: (0, 0))
    return pl.pallas_call(
        _dual_body,
        grid=(grid,),
        in_specs=[row_spec, row_spec, full_spec, full_spec, bias_spec],
        out_specs=[row_spec, row_spec],
        out_shape=[jax.ShapeDtypeStruct((N, H), jnp.float32)] * 2,
    )


_dual = _make_dual()


def _dec_out_body(s_ref, w2_ref, b2_ref, o_ref):
    s = jnp.maximum(s_ref[...], 0.0)
    o_ref[...] = jnp.sum(s * w2_ref[...], axis=1, keepdims=True) + b2_ref[...]


def _make_dec_out():
    blk = 2176
    grid = LP // blk
    return pl.pallas_call(
        _dec_out_body,
        grid=(grid,),
        in_specs=[pl.BlockSpec((blk, H), lambda i: (i, 0)),
                  pl.BlockSpec((1, H), lambda i: (0, 0)),
                  pl.BlockSpec((1, 1), lambda i: (0, 0))],
        out_specs=pl.BlockSpec((blk, 1), lambda i: (i, 0)),
        out_shape=jax.ShapeDtypeStruct((LP, 1), jnp.float32),
    )


_dec_out = _make_dec_out()


def kernel(x_drug, x_protein, edge_index_drug_protein, edge_index_protein_drug,
           edge_label_index, Wl1_dp, bl1_dp, Wr1_dp, Wl1_pd, bl1_pd, Wr1_pd,
           Wl2_dp, bl2_dp, Wr2_dp, Wl2_pd, bl2_pd, Wr2_pd,
           Wdec1, bdec1, Wdec2, bdec2):
    def pack_edges(ei):
        src = jnp.concatenate(
            [ei[0], jnp.zeros((PE - E,), ei.dtype)]).reshape(NCHP, 1, CH)
        dst = jnp.concatenate(
            [ei[1], jnp.full((PE - E,), N, ei.dtype)]).reshape(NCHP, 1, CH)
        return jnp.concatenate([src, dst], axis=1)

    idx_dp = pack_edges(edge_index_drug_protein)
    idx_pd = pack_edges(edge_index_protein_drug)
    pad = jnp.zeros((LP - L,), edge_label_index.dtype)
    row = jnp.concatenate([edge_label_index[0], pad])
    col = jnp.concatenate([edge_label_index[1], pad])

    b1 = bl1_dp.reshape(1, H)
    b2 = bl1_pd.reshape(1, H)
    b3 = bl2_dp.reshape(1, H)
    b4 = bl2_pd.reshape(1, H)

    tbl_dp, init_prot, tbl_pd, init_drug = _quad_plain(
        x_drug, x_protein, Wl1_dp, Wr1_dp, Wl1_pd, Wr1_pd, b1, b2)
    hpre_prot, hpre_drug = _seg_pair(
        idx_dp, tbl_dp, init_prot, idx_pd, tbl_pd, init_drug)

    tbl2_dp, init2_prot, tbl2_pd, init2_drug = _quad_relu(
        hpre_drug, hpre_prot, Wl2_dp, Wr2_dp, Wl2_pd, Wr2_pd, b3, b4)
    z_prot, z_drug = _seg_pair(
        idx_dp, tbl2_dp, init2_prot, idx_pd, tbl2_pd, init2_drug)

    u_drug, u_prot = _dual(z_drug, z_prot, Wdec1[:H], Wdec1[H:],
                           bdec1.reshape(1, H))
    s = _dec_gather(u_drug, u_prot, row, col)
    out2 = _dec_out(s, Wdec2.reshape(1, H), bdec2.reshape(1, 1))
    out = out2.reshape(-1)[:L]
    return (z_drug, z_prot, out)


# speedup vs baseline: 2.2084x; 2.2084x over previous
import functools

import jax
import jax.numpy as jnp
from jax import lax
from jax.experimental import pallas as pl
from jax.experimental.pallas import tpu as pltpu
from jax.experimental.pallas import tpu_sc as plsc

NC, NS = 2, 16
NW = NC * NS
N = 10000
H = 128
E = 320000
L = 100000
CH = 128
NCHUNK = E // CH
SEG_STEPS = 79
LP = 100096
LCHUNK = LP // CH
K_DEC = -(-LCHUNK // NW)
RC = 400
NRC = N // RC
K_RC = -(-NRC // NS)

_MESH = plsc.VectorSubcoreMesh(
    core_axis_name="c", subcore_axis_name="s", num_cores=NC, num_subcores=NS)


def _seg_pair_body(src_a, dst_a, tbl_a, init_a, src_b, dst_b, tbl_b, init_b,
                   out_a, out_b, *scr):
    sidx = scr[0:2]
    didx = scr[2:4]
    rows = scr[4:6]
    acc = scr[6]
    semg = scr[7:9]
    c = lax.axis_index("c")
    t = lax.axis_index("s")

    def rows_loop(body_fn):
        def body(k, carry):
            cid = k * NS + t

            @pl.when(cid < NRC)
            def _():
                body_fn(cid * RC)

            return carry

        lax.fori_loop(0, K_RC, body, 0)

    def run(src_h, dst_h, tbl_h, init_h, out_h):
        def prep(i, j):
            cid = i * NS + t

            @pl.when(cid < NCHUNK)
            def _():
                base = cid * CH
                pltpu.sync_copy(src_h.at[pl.ds(base, CH)], sidx[j])
                pltpu.sync_copy(dst_h.at[pl.ds(base, CH)], didx[j])
                pltpu.async_copy(tbl_h.at[sidx[j]], rows[j], semg[j])

        def drain(i, j):
            cid = i * NS + t

            @pl.when(cid < NCHUNK)
            def _():
                pltpu.make_async_copy(
                    tbl_h.at[sidx[j]], rows[j], semg[j]).wait()
                pltpu.sync_copy(rows[j], acc.at[didx[j]], add=True)

        rows_loop(lambda r0: pltpu.sync_copy(
            init_h.at[pl.ds(r0, RC)], acc.at[pl.ds(r0, RC)]))
        prep(0, 0)
        plsc.subcore_barrier()

        def pair(kk, carry):
            i0 = 2 * kk
            prep(i0 + 1, 1)
            drain(i0, 0)
            prep(i0 + 2, 0)
            drain(i0 + 1, 1)
            return carry

        lax.fori_loop(0, SEG_STEPS, pair, 0)
        plsc.subcore_barrier()
        rows_loop(lambda r0: pltpu.sync_copy(
            acc.at[pl.ds(r0, RC)], out_h.at[pl.ds(r0, RC)]))

    @pl.when(c == 0)
    def _():
        run(src_a, dst_a, tbl_a, init_a, out_a)

    @pl.when(c == 1)
    def _():
        run(src_b, dst_b, tbl_b, init_b, out_b)


_seg_pair = pl.kernel(
    _seg_pair_body,
    out_type=(jax.ShapeDtypeStruct((N, H), jnp.float32),
              jax.ShapeDtypeStruct((N, H), jnp.float32)),
    mesh=_MESH,
    scratch_types=(
        [pltpu.VMEM((CH,), jnp.int32)] * 4
        + [pltpu.VMEM((CH, H), jnp.float32)] * 2
        + [pltpu.VMEM_SHARED((N, H), jnp.float32)]
        + [pltpu.SemaphoreType.DMA] * 2
    ),
)


def _dec_gather_body(ud, up, ridx_h, cidx_h, s_h, idx_v, buf, sem):
    wid = lax.axis_index("c") * NS + lax.axis_index("s")

    def body(k, carry):
        cid = k * NW + wid

        @pl.when(cid < LCHUNK)
        def _():
            base = cid * CH
            pltpu.sync_copy(ridx_h.at[pl.ds(base, CH)], idx_v)
            pltpu.async_copy(ud.at[idx_v], buf, sem).wait()
            pltpu.sync_copy(cidx_h.at[pl.ds(base, CH)], idx_v)
            pltpu.async_copy(up.at[idx_v], buf, sem, add=True).wait()
            pltpu.sync_copy(buf, s_h.at[pl.ds(base, CH)])

        return carry

    lax.fori_loop(0, K_DEC, body, 0)


_dec_gather = pl.kernel(
    _dec_gather_body,
    out_type=jax.ShapeDtypeStruct((LP, H), jnp.float32),
    mesh=_MESH,
    scratch_types=[
        pltpu.VMEM((CH,), jnp.int32),
        pltpu.VMEM((CH, H), jnp.float32),
        pltpu.SemaphoreType.DMA,
    ],
)


def _quad_body(a_ref, b_ref, w1, w2, w3, w4, bias1, bias2,
               o1, o2, o3, o4, *, relu):
    a = a_ref[...]
    b = b_ref[...]
    if relu:
        a = jnp.maximum(a, 0.0)
        b = jnp.maximum(b, 0.0)
    f32 = jnp.float32
    o1[...] = jnp.dot(a, w1[...], preferred_element_type=f32)
    o2[...] = jnp.dot(b, w2[...], preferred_element_type=f32) + bias1[...]
    o3[...] = jnp.dot(b, w3[...], preferred_element_type=f32)
    o4[...] = jnp.dot(a, w4[...], preferred_element_type=f32) + bias2[...]


def _make_quad(relu):
    blk = 1000
    grid = N // blk
    row_spec = pl.BlockSpec((blk, H), lambda i: (i, 0))
    full_spec = pl.BlockSpec((H, H), lambda i: (0, 0))
    bias_spec = pl.BlockSpec((1, H), lambda i: (0, 0))
    return pl.pallas_call(
        functools.partial(_quad_body, relu=relu),
        grid=(grid,),
        in_specs=[row_spec, row_spec, full_spec, full_spec, full_spec,
                  full_spec, bias_spec, bias_spec],
        out_specs=[row_spec, row_spec, row_spec, row_spec],
        out_shape=[jax.ShapeDtypeStruct((N, H), jnp.float32)] * 4,
    )


_quad_plain = _make_quad(relu=False)
_quad_relu = _make_quad(relu=True)


def _dual_body(a_ref, b_ref, w1, w2, bias1, o1, o2):
    f32 = jnp.float32
    o1[...] = jnp.dot(a_ref[...], w1[...], preferred_element_type=f32) + bias1[...]
    o2[...] = jnp.dot(b_ref[...], w2[...], preferred_element_type=f32)


def _make_dual():
    blk = 1000
    grid = N // blk
    row_spec = pl.BlockSpec((blk, H), lambda i: (i, 0))
    full_spec = pl.BlockSpec((H, H), lambda i: (0, 0))
    bias_spec = pl.BlockSpec((1, H), lambda i: (0, 0))
    return pl.pallas_call(
        _dual_body,
        grid=(grid,),
        in_specs=[row_spec, row_spec, full_spec, full_spec, bias_spec],
        out_specs=[row_spec, row_spec],
        out_shape=[jax.ShapeDtypeStruct((N, H), jnp.float32)] * 2,
    )


_dual = _make_dual()


def _dec_out_body(s_ref, w2_ref, b2_ref, o_ref):
    s = jnp.maximum(s_ref[...], 0.0)
    o_ref[...] = jnp.sum(s * w2_ref[...], axis=1, keepdims=True) + b2_ref[...]


def _make_dec_out():
    blk = 2176
    grid = LP // blk
    return pl.pallas_call(
        _dec_out_body,
        grid=(grid,),
        in_specs=[pl.BlockSpec((blk, H), lambda i: (i, 0)),
                  pl.BlockSpec((1, H), lambda i: (0, 0)),
                  pl.BlockSpec((1, 1), lambda i: (0, 0))],
        out_specs=pl.BlockSpec((blk, 1), lambda i: (i, 0)),
        out_shape=jax.ShapeDtypeStruct((LP, 1), jnp.float32),
    )


_dec_out = _make_dec_out()


def kernel(x_drug, x_protein, edge_index_drug_protein, edge_index_protein_drug,
           edge_label_index, Wl1_dp, bl1_dp, Wr1_dp, Wl1_pd, bl1_pd, Wr1_pd,
           Wl2_dp, bl2_dp, Wr2_dp, Wl2_pd, bl2_pd, Wr2_pd,
           Wdec1, bdec1, Wdec2, bdec2):
    src_dp = edge_index_drug_protein[0]
    dst_dp = edge_index_drug_protein[1]
    src_pd = edge_index_protein_drug[0]
    dst_pd = edge_index_protein_drug[1]
    pad = jnp.zeros((LP - L,), edge_label_index.dtype)
    row = jnp.concatenate([edge_label_index[0], pad])
    col = jnp.concatenate([edge_label_index[1], pad])

    b1 = bl1_dp.reshape(1, H)
    b2 = bl1_pd.reshape(1, H)
    b3 = bl2_dp.reshape(1, H)
    b4 = bl2_pd.reshape(1, H)

    tbl_dp, init_prot, tbl_pd, init_drug = _quad_plain(
        x_drug, x_protein, Wl1_dp, Wr1_dp, Wl1_pd, Wr1_pd, b1, b2)
    hpre_prot, hpre_drug = _seg_pair(
        src_dp, dst_dp, tbl_dp, init_prot, src_pd, dst_pd, tbl_pd, init_drug)

    tbl2_dp, init2_prot, tbl2_pd, init2_drug = _quad_relu(
        hpre_drug, hpre_prot, Wl2_dp, Wr2_dp, Wl2_pd, Wr2_pd, b3, b4)
    z_prot, z_drug = _seg_pair(
        src_dp, dst_dp, tbl2_dp, init2_prot, src_pd, dst_pd, tbl2_pd,
        init2_drug)

    u_drug, u_prot = _dual(z_drug, z_prot, Wdec1[:H], Wdec1[H:],
                           bdec1.reshape(1, H))
    s = _dec_gather(u_drug, u_prot, row, col)
    out2 = _dec_out(s, Wdec2.reshape(1, H), bdec2.reshape(1, 1))
    out = out2.reshape(-1)[:L]
    return (z_drug, z_prot, out)


# speedup vs baseline: 2.3316x; 1.0558x over previous
import functools

import jax
import jax.numpy as jnp
from jax import lax
from jax.experimental import pallas as pl
from jax.experimental.pallas import tpu as pltpu
from jax.experimental.pallas import tpu_sc as plsc

NC, NS = 2, 16
NW = NC * NS
N = 10000
H = 128
E = 320000
L = 100000
CH = 128
NCHUNK = E // CH
SEG_STEPS = 79
LP = 100096
LCHUNK = LP // CH
DEC_STEPS = 13
RC = 400
NRC = N // RC
K_RC = -(-NRC // NS)

_MESH = plsc.VectorSubcoreMesh(
    core_axis_name="c", subcore_axis_name="s", num_cores=NC, num_subcores=NS)


def _seg_pair_body(edg_a, tbl_a, init_a, edg_b, tbl_b, init_b,
                   out_a, out_b, *scr):
    sidx = scr[0:2]
    didx = scr[2:4]
    rows = scr[4:6]
    acc = scr[6]
    semg = scr[7:9]
    c = lax.axis_index("c")
    t = lax.axis_index("s")

    def rows_loop(body_fn):
        def body(k, carry):
            cid = k * NS + t

            @pl.when(cid < NRC)
            def _():
                body_fn(cid * RC)

            return carry

        lax.fori_loop(0, K_RC, body, 0)

    def run(edg_h, tbl_h, init_h, out_h):
        def prep(i, j):
            cid = i * NS + t

            @pl.when(cid < NCHUNK)
            def _():
                base = cid * CH
                pltpu.sync_copy(edg_h.at[pl.ds(base, CH)], sidx[j])
                pltpu.sync_copy(edg_h.at[pl.ds(E + base, CH)], didx[j])
                pltpu.async_copy(tbl_h.at[sidx[j]], rows[j], semg[j])

        def drain(i, j):
            cid = i * NS + t

            @pl.when(cid < NCHUNK)
            def _():
                pltpu.make_async_copy(
                    tbl_h.at[sidx[j]], rows[j], semg[j]).wait()
                pltpu.sync_copy(rows[j], acc.at[didx[j]], add=True)

        rows_loop(lambda r0: pltpu.sync_copy(
            init_h.at[pl.ds(r0, RC)], acc.at[pl.ds(r0, RC)]))
        prep(0, 0)
        plsc.subcore_barrier()

        def pair(kk, carry):
            i0 = 2 * kk
            prep(i0 + 1, 1)
            drain(i0, 0)
            prep(i0 + 2, 0)
            drain(i0 + 1, 1)
            return carry

        lax.fori_loop(0, SEG_STEPS, pair, 0)
        plsc.subcore_barrier()
        rows_loop(lambda r0: pltpu.sync_copy(
            acc.at[pl.ds(r0, RC)], out_h.at[pl.ds(r0, RC)]))

    @pl.when(c == 0)
    def _():
        run(edg_a, tbl_a, init_a, out_a)

    @pl.when(c == 1)
    def _():
        run(edg_b, tbl_b, init_b, out_b)


_seg_pair = pl.kernel(
    _seg_pair_body,
    out_type=(jax.ShapeDtypeStruct((N, H), jnp.float32),
              jax.ShapeDtypeStruct((N, H), jnp.float32)),
    mesh=_MESH,
    scratch_types=(
        [pltpu.VMEM((CH,), jnp.int32)] * 4
        + [pltpu.VMEM((CH, H), jnp.float32)] * 2
        + [pltpu.VMEM_SHARED((N, H), jnp.float32)]
        + [pltpu.SemaphoreType.DMA] * 2
    ),
)


def _dec_gather_body(ud, up, lbl_h, s_h, *scr):
    ridx = scr[0:2]
    cidx = scr[2:4]
    buf = scr[4:6]
    sem_a = scr[6:8]
    sem_b = scr[8:10]
    wid = lax.axis_index("c") * NS + lax.axis_index("s")

    def prep(i, j):
        cid = i * NW + wid

        @pl.when(cid < LCHUNK)
        def _():
            pltpu.sync_copy(lbl_h.at[pl.ds(cid * CH, CH)], ridx[j])
            pltpu.async_copy(ud.at[ridx[j]], buf[j], sem_a[j])

    def mid(i, j):
        cid = i * NW + wid

        @pl.when(cid < LCHUNK)
        def _():
            pltpu.make_async_copy(ud.at[ridx[j]], buf[j], sem_a[j]).wait()
            pltpu.sync_copy(lbl_h.at[pl.ds(L + cid * CH, CH)], cidx[j])
            pltpu.async_copy(up.at[cidx[j]], buf[j], sem_b[j], add=True)

    def fin(i, j):
        cid = i * NW + wid

        @pl.when(cid < LCHUNK)
        def _():
            pltpu.make_async_copy(up.at[cidx[j]], buf[j], sem_b[j]).wait()
            pltpu.sync_copy(buf[j], s_h.at[pl.ds(cid * CH, CH)])

    prep(0, 0)

    def pair(kk, carry):
        i0 = 2 * kk
        prep(i0 + 1, 1)
        mid(i0, 0)
        fin(i0, 0)
        prep(i0 + 2, 0)
        mid(i0 + 1, 1)
        fin(i0 + 1, 1)
        return carry

    lax.fori_loop(0, DEC_STEPS, pair, 0)


_dec_gather = pl.kernel(
    _dec_gather_body,
    out_type=jax.ShapeDtypeStruct((LP, H), jnp.float32),
    mesh=_MESH,
    scratch_types=(
        [pltpu.VMEM((CH,), jnp.int32)] * 4
        + [pltpu.VMEM((CH, H), jnp.float32)] * 2
        + [pltpu.SemaphoreType.DMA] * 4
    ),
)


def _quad_body(a_ref, b_ref, w1, w2, w3, w4, bias1, bias2,
               o1, o2, o3, o4, *, relu):
    a = a_ref[...]
    b = b_ref[...]
    if relu:
        a = jnp.maximum(a, 0.0)
        b = jnp.maximum(b, 0.0)
    f32 = jnp.float32
    o1[...] = jnp.dot(a, w1[...], preferred_element_type=f32)
    o2[...] = jnp.dot(b, w2[...], preferred_element_type=f32) + bias1[...]
    o3[...] = jnp.dot(b, w3[...], preferred_element_type=f32)
    o4[...] = jnp.dot(a, w4[...], preferred_element_type=f32) + bias2[...]


def _make_quad(relu):
    blk = 1000
    grid = N // blk
    row_spec = pl.BlockSpec((blk, H), lambda i: (i, 0))
    full_spec = pl.BlockSpec((H, H), lambda i: (0, 0))
    bias_spec = pl.BlockSpec((1, H), lambda i: (0, 0))
    return pl.pallas_call(
        functools.partial(_quad_body, relu=relu),
        grid=(grid,),
        in_specs=[row_spec, row_spec, full_spec, full_spec, full_spec,
                  full_spec, bias_spec, bias_spec],
        out_specs=[row_spec, row_spec, row_spec, row_spec],
        out_shape=[jax.ShapeDtypeStruct((N, H), jnp.float32)] * 4,
    )


_quad_plain = _make_quad(relu=False)
_quad_relu = _make_quad(relu=True)


def _dual_body(a_ref, b_ref, w1, w2, bias1, o1, o2):
    f32 = jnp.float32
    o1[...] = jnp.dot(a_ref[...], w1[...], preferred_element_type=f32) + bias1[...]
    o2[...] = jnp.dot(b_ref[...], w2[...], preferred_element_type=f32)


def _make_dual():
    blk = 1000
    grid = N // blk
    row_spec = pl.BlockSpec((blk, H), lambda i: (i, 0))
    full_spec = pl.BlockSpec((H, H), lambda i: (0, 0))
    bias_spec = pl.BlockSpec((1, H), lambda i: (0, 0))
    return pl.pallas_call(
        _dual_body,
        grid=(grid,),
        in_specs=[row_spec, row_spec, full_spec, full_spec, bias_spec],
        out_specs=[row_spec, row_spec],
        out_shape=[jax.ShapeDtypeStruct((N, H), jnp.float32)] * 2,
    )


_dual = _make_dual()


def _dec_out_body(s_ref, w2_ref, b2_ref, o_ref):
    s = jnp.maximum(s_ref[...], 0.0)
    o_ref[...] = jnp.sum(s * w2_ref[...], axis=1, keepdims=True) + b2_ref[...]


def _make_dec_out():
    blk = 2176
    grid = LP // blk
    return pl.pallas_call(
        _dec_out_body,
        grid=(grid,),
        in_specs=[pl.BlockSpec((blk, H), lambda i: (i, 0)),
                  pl.BlockSpec((1, H), lambda i: (0, 0)),
                  pl.BlockSpec((1, 1), lambda i: (0, 0))],
        out_specs=pl.BlockSpec((blk, 1), lambda i: (i, 0)),
        out_shape=jax.ShapeDtypeStruct((LP, 1), jnp.float32),
    )


_dec_out = _make_dec_out()


def kernel(x_drug, x_protein, edge_index_drug_protein, edge_index_protein_drug,
           edge_label_index, Wl1_dp, bl1_dp, Wr1_dp, Wl1_pd, bl1_pd, Wr1_pd,
           Wl2_dp, bl2_dp, Wr2_dp, Wl2_pd, bl2_pd, Wr2_pd,
           Wdec1, bdec1, Wdec2, bdec2):
    edg_dp = edge_index_drug_protein.reshape(-1)
    edg_pd = edge_index_protein_drug.reshape(-1)
    lbl = jnp.concatenate(
        [edge_label_index.reshape(-1),
         jnp.zeros((LP - L,), edge_label_index.dtype)])

    b1 = bl1_dp.reshape(1, H)
    b2 = bl1_pd.reshape(1, H)
    b3 = bl2_dp.reshape(1, H)
    b4 = bl2_pd.reshape(1, H)

    tbl_dp, init_prot, tbl_pd, init_drug = _quad_plain(
        x_drug, x_protein, Wl1_dp, Wr1_dp, Wl1_pd, Wr1_pd, b1, b2)
    hpre_prot, hpre_drug = _seg_pair(
        edg_dp, tbl_dp, init_prot, edg_pd, tbl_pd, init_drug)

    tbl2_dp, init2_prot, tbl2_pd, init2_drug = _quad_relu(
        hpre_drug, hpre_prot, Wl2_dp, Wr2_dp, Wl2_pd, Wr2_pd, b3, b4)
    z_prot, z_drug = _seg_pair(
        edg_dp, tbl2_dp, init2_prot, edg_pd, tbl2_pd, init2_drug)

    u_drug, u_prot = _dual(z_drug, z_prot, Wdec1[:H], Wdec1[H:],
                           bdec1.reshape(1, H))
    s = _dec_gather(u_drug, u_prot, lbl)
    out2 = _dec_out(s, Wdec2.reshape(1, H), bdec2.reshape(1, 1))
    out = out2.reshape(-1)[:L]
    return (z_drug, z_prot, out)
